# Initial kernel scaffold; baseline (speedup 1.0000x reference)
import functools

import jax
import jax.numpy as jnp
from jax import lax
from jax.experimental import pallas as pl
from jax.experimental.pallas import tpu as pltpu
from jax.experimental.pallas import tpu_sc as plsc

N_SO = 100000
DIM = 32
BATCH = 16384
N_ELEC = 50

NC = 2
NS = 16
NW = NC * NS
B_PER_W = BATCH // NW
CHUNK_IDX = 2 * N_ELEC
N_CHUNKS = B_PER_W * N_ELEC // CHUNK_IDX


def _accum_rows(buf, out_v, b_v, out_row):
    for r in range(2):
        base = r * N_ELEC
        for h in range(2):
            sl = pl.ds(16 * h, 16)
            acc = [buf[base + k, sl] for k in range(4)]
            for e in range(4, N_ELEC):
                acc[e % 4] = acc[e % 4] + buf[base + e, sl]
            out_v[out_row + r, sl] = ((acc[0] + acc[1]) + (acc[2] + acc[3])) + b_v[sl]


@functools.partial(
    pl.kernel,
    out_type=jax.ShapeDtypeStruct((BATCH, DIM), jnp.float32),
    mesh=plsc.VectorSubcoreMesh(core_axis_name="c", subcore_axis_name="s"),
    scratch_types=[
        pltpu.VMEM((N_CHUNKS, CHUNK_IDX), jnp.int32),
        pltpu.VMEM((CHUNK_IDX, DIM), jnp.float32),
        pltpu.VMEM((CHUNK_IDX, DIM), jnp.float32),
        pltpu.VMEM((B_PER_W, DIM), jnp.float32),
        pltpu.VMEM((DIM,), jnp.float32),
        pltpu.SemaphoreType.DMA,
        pltpu.SemaphoreType.DMA,
    ],
)
def _pool_kernel(occ_hbm, e_hbm, b_hbm, out_hbm,
                 idx_v, buf0, buf1, out_v, b_v, sem0, sem1):
    wid = lax.axis_index("s") * NC + lax.axis_index("c")

    pltpu.sync_copy(b_hbm, b_v)
    pltpu.sync_copy(occ_hbm.at[pl.ds(wid * N_CHUNKS, N_CHUNKS), :], idx_v)

    pltpu.async_copy(e_hbm.at[idx_v.at[0]], buf0, sem0)
    pltpu.async_copy(e_hbm.at[idx_v.at[1]], buf1, sem1)

    def body(j2, carry):
        c = j2 * 2
        pltpu.make_async_copy(e_hbm.at[idx_v.at[c]], buf0, sem0).wait()
        _accum_rows(buf0, out_v, b_v, c * 2)

        @pl.when(j2 < N_CHUNKS // 2 - 1)
        def _():
            pltpu.async_copy(e_hbm.at[idx_v.at[c + 2]], buf0, sem0)

        pltpu.make_async_copy(e_hbm.at[idx_v.at[c + 1]], buf1, sem1).wait()
        _accum_rows(buf1, out_v, b_v, c * 2 + 2)

        @pl.when(j2 < N_CHUNKS // 2 - 1)
        def _():
            pltpu.async_copy(e_hbm.at[idx_v.at[c + 3]], buf1, sem1)

        return carry

    lax.fori_loop(0, N_CHUNKS // 2, body, 0)

    pltpu.sync_copy(out_v, out_hbm.at[pl.ds(wid * B_PER_W, B_PER_W), :])


def kernel(occ_so, E, b):
    occ = occ_so.astype(jnp.int32).reshape(BATCH * N_ELEC // CHUNK_IDX, CHUNK_IDX)
    return _pool_kernel(occ, E, b)


# speedup vs baseline: 16.6116x; 16.6116x over previous
import functools

import jax
import jax.numpy as jnp
from jax import lax
from jax.experimental import pallas as pl
from jax.experimental.pallas import tpu as pltpu
from jax.experimental.pallas import tpu_sc as plsc

N_SO = 100000
DIM = 32
BATCH = 16384
N_ELEC = 50

NC = 2
NS = 16
NW = NC * NS
B_PER_W = BATCH // NW
CHUNK_IDX = 2 * N_ELEC
N_CHUNKS = B_PER_W * N_ELEC // CHUNK_IDX


def _accum_rows(buf, out_v, b_v, out_row):
    for r in range(2):
        base = r * N_ELEC
        for h in range(2):
            sl = pl.ds(16 * h, 16)
            acc = [buf[base + k, sl] for k in range(4)]
            for e in range(4, N_ELEC):
                acc[e % 4] = acc[e % 4] + buf[base + e, sl]
            out_v[out_row + r, sl] = ((acc[0] + acc[1]) + (acc[2] + acc[3])) + b_v[sl]


@functools.partial(
    pl.kernel,
    out_type=jax.ShapeDtypeStruct((BATCH, DIM), jnp.float32),
    mesh=plsc.VectorSubcoreMesh(core_axis_name="c", subcore_axis_name="s"),
    compiler_params=pltpu.CompilerParams(use_tc_tiling_on_sc=False),
    scratch_types=[
        pltpu.VMEM((N_CHUNKS, CHUNK_IDX), jnp.int32),
        pltpu.VMEM((CHUNK_IDX, DIM), jnp.float32),
        pltpu.VMEM((CHUNK_IDX, DIM), jnp.float32),
        pltpu.VMEM((B_PER_W, DIM), jnp.float32),
        pltpu.VMEM((DIM,), jnp.float32),
        pltpu.SemaphoreType.DMA,
        pltpu.SemaphoreType.DMA,
    ],
)
def _pool_kernel(occ_hbm, e_hbm, b_hbm, out_hbm,
                 idx_v, buf0, buf1, out_v, b_v, sem0, sem1):
    wid = lax.axis_index("s") * NC + lax.axis_index("c")

    pltpu.sync_copy(b_hbm, b_v)
    pltpu.sync_copy(occ_hbm.at[pl.ds(wid * N_CHUNKS, N_CHUNKS), :], idx_v)

    pltpu.async_copy(e_hbm.at[idx_v.at[0]], buf0, sem0)
    pltpu.async_copy(e_hbm.at[idx_v.at[1]], buf1, sem1)

    def body(j2, carry):
        c = j2 * 2
        pltpu.make_async_copy(e_hbm.at[idx_v.at[c]], buf0, sem0).wait()
        _accum_rows(buf0, out_v, b_v, c * 2)

        @pl.when(j2 < N_CHUNKS // 2 - 1)
        def _():
            pltpu.async_copy(e_hbm.at[idx_v.at[c + 2]], buf0, sem0)

        pltpu.make_async_copy(e_hbm.at[idx_v.at[c + 1]], buf1, sem1).wait()
        _accum_rows(buf1, out_v, b_v, c * 2 + 2)

        @pl.when(j2 < N_CHUNKS // 2 - 1)
        def _():
            pltpu.async_copy(e_hbm.at[idx_v.at[c + 3]], buf1, sem1)

        return carry

    lax.fori_loop(0, N_CHUNKS // 2, body, 0)

    pltpu.sync_copy(out_v, out_hbm.at[pl.ds(wid * B_PER_W, B_PER_W), :])


def kernel(occ_so, E, b):
    occ = occ_so.astype(jnp.int32).reshape(BATCH * N_ELEC // CHUNK_IDX, CHUNK_IDX)
    return _pool_kernel(occ, E, b)


# speedup vs baseline: 22.1134x; 1.3312x over previous
import functools

import jax
import jax.numpy as jnp
from jax import lax
from jax.experimental import pallas as pl
from jax.experimental.pallas import tpu as pltpu
from jax.experimental.pallas import tpu_sc as plsc

N_SO = 100000
DIM = 32
BATCH = 16384
N_ELEC = 50

NC = 2
NS = 16
NW = NC * NS
B_PER_W = BATCH // NW
CHUNK_IDX = 2 * N_ELEC
N_CHUNKS = B_PER_W * N_ELEC // CHUNK_IDX


def _accum_rows(buf, out_v, b_v, out_row):
    for r in range(2):
        base = r * N_ELEC
        for h in range(2):
            sl = pl.ds(16 * h, 16)
            acc = [buf[base + k, sl] for k in range(4)]
            for e in range(4, N_ELEC):
                acc[e % 4] = acc[e % 4] + buf[base + e, sl]
            out_v[out_row + r, sl] = ((acc[0] + acc[1]) + (acc[2] + acc[3])) + b_v[sl]


@functools.partial(
    pl.kernel,
    out_type=jax.ShapeDtypeStruct((BATCH, DIM), jnp.float32),
    mesh=plsc.VectorSubcoreMesh(core_axis_name="c", subcore_axis_name="s"),
    compiler_params=pltpu.CompilerParams(use_tc_tiling_on_sc=False),
    scratch_types=[
        pltpu.VMEM((N_CHUNKS, CHUNK_IDX), jnp.int32),
        pltpu.VMEM((CHUNK_IDX, DIM), jnp.float32),
        pltpu.VMEM((CHUNK_IDX, DIM), jnp.float32),
        pltpu.VMEM((CHUNK_IDX, DIM), jnp.float32),
        pltpu.VMEM((CHUNK_IDX, DIM), jnp.float32),
        pltpu.VMEM((B_PER_W, DIM), jnp.float32),
        pltpu.VMEM((DIM,), jnp.float32),
        pltpu.SemaphoreType.DMA,
        pltpu.SemaphoreType.DMA,
        pltpu.SemaphoreType.DMA,
        pltpu.SemaphoreType.DMA,
    ],
)
def _pool_kernel(occ_hbm, e_hbm, b_hbm, out_hbm,
                 idx_v, buf0, buf1, buf2, buf3, out_v, b_v,
                 sem0, sem1, sem2, sem3):
    wid = lax.axis_index("s") * NC + lax.axis_index("c")
    bufs = (buf0, buf1, buf2, buf3)
    sems = (sem0, sem1, sem2, sem3)
    nbuf = 4

    pltpu.sync_copy(b_hbm, b_v)
    pltpu.sync_copy(occ_hbm.at[pl.ds(wid * N_CHUNKS, N_CHUNKS), :], idx_v)

    for k in range(nbuf):
        pltpu.async_copy(e_hbm.at[idx_v.at[k]], bufs[k], sems[k])

    def body(j, carry):
        c = j * nbuf
        for k in range(nbuf):
            pltpu.make_async_copy(e_hbm.at[idx_v.at[c + k]], bufs[k], sems[k]).wait()
            _accum_rows(bufs[k], out_v, b_v, (c + k) * 2)

            @pl.when(c + k + nbuf < N_CHUNKS)
            def _():
                pltpu.async_copy(e_hbm.at[idx_v.at[c + k + nbuf]], bufs[k], sems[k])

        return carry

    lax.fori_loop(0, N_CHUNKS // nbuf, body, 0)

    pltpu.sync_copy(out_v, out_hbm.at[pl.ds(wid * B_PER_W, B_PER_W), :])


def kernel(occ_so, E, b):
    occ = occ_so.astype(jnp.int32).reshape(BATCH * N_ELEC // CHUNK_IDX, CHUNK_IDX)
    return _pool_kernel(occ, E, b)


# speedup vs baseline: 22.7648x; 1.0295x over previous
import functools

import jax
import jax.numpy as jnp
from jax import lax
from jax.experimental import pallas as pl
from jax.experimental.pallas import tpu as pltpu
from jax.experimental.pallas import tpu_sc as plsc

N_SO = 100000
DIM = 32
BATCH = 16384
N_ELEC = 50

NC = 2
NS = 16
NW = NC * NS
B_PER_W = BATCH // NW
NBUF = 8


def _accum_row(buf, out_v, b_v, out_row):
    for h in range(2):
        sl = pl.ds(16 * h, 16)
        acc = [buf[k, sl] for k in range(4)]
        for e in range(4, N_ELEC):
            acc[e % 4] = acc[e % 4] + buf[e, sl]
        out_v[out_row, sl] = ((acc[0] + acc[1]) + (acc[2] + acc[3])) + b_v[sl]


@functools.partial(
    pl.kernel,
    out_type=jax.ShapeDtypeStruct((BATCH, DIM), jnp.float32),
    mesh=plsc.VectorSubcoreMesh(core_axis_name="c", subcore_axis_name="s"),
    compiler_params=pltpu.CompilerParams(use_tc_tiling_on_sc=False),
    scratch_types=(
        [pltpu.VMEM((B_PER_W, N_ELEC), jnp.int32)]
        + [pltpu.VMEM((N_ELEC, DIM), jnp.float32)] * NBUF
        + [pltpu.VMEM((B_PER_W, DIM), jnp.float32)]
        + [pltpu.VMEM((DIM,), jnp.float32)]
        + [pltpu.SemaphoreType.DMA] * NBUF
    ),
)
def _pool_kernel(occ_hbm, e_hbm, b_hbm, out_hbm, idx_v, *rest):
    bufs = rest[:NBUF]
    out_v = rest[NBUF]
    b_v = rest[NBUF + 1]
    sems = rest[NBUF + 2:]

    wid = lax.axis_index("s") * NC + lax.axis_index("c")

    pltpu.sync_copy(b_hbm, b_v)
    pltpu.sync_copy(occ_hbm.at[pl.ds(wid * B_PER_W, B_PER_W), :], idx_v)

    for k in range(NBUF):
        pltpu.async_copy(e_hbm.at[idx_v.at[k]], bufs[k], sems[k])

    def body(j, carry):
        r = j * NBUF
        for k in range(NBUF):
            pltpu.make_async_copy(e_hbm.at[idx_v.at[r + k]], bufs[k], sems[k]).wait()
            _accum_row(bufs[k], out_v, b_v, r + k)

            @pl.when(r + k + NBUF < B_PER_W)
            def _():
                pltpu.async_copy(e_hbm.at[idx_v.at[r + k + NBUF]], bufs[k], sems[k])

        return carry

    lax.fori_loop(0, B_PER_W // NBUF, body, 0)

    pltpu.sync_copy(out_v, out_hbm.at[pl.ds(wid * B_PER_W, B_PER_W), :])


def kernel(occ_so, E, b):
    return _pool_kernel(occ_so.astype(jnp.int32), E, b)
